# baseline (speedup 1.0000x reference)
import functools

import jax
import jax.numpy as jnp
from jax import lax
from jax.experimental import pallas as pl
from jax.experimental.pallas import tpu as pltpu
from jax.experimental.pallas import tpu_sc as plsc

B = 16384
D = 64
PACK = 8
PW = PACK * D

_info = plsc.get_sparse_core_info()
NC, NS = _info.num_cores, _info.num_subcores
NW = NC * NS
BPW = B // NW
CHUNK = 64
NCH = BPW // CHUNK


def _sc_gather_body(utab, uidx, mtab, midx, uout, mout,
                    uidx_v, midx_v, rows, ebuf, sem):
    wid = lax.axis_index("s") * NC + lax.axis_index("c")
    base = wid * BPW
    pltpu.sync_copy(uidx.at[pl.ds(base, BPW)], uidx_v.at[pl.ds(0, BPW)])
    pltpu.sync_copy(midx.at[pl.ds(base, BPW)], midx_v.at[pl.ds(0, BPW)])
    for tab, idx_s, out in ((utab, uidx_v, uout), (mtab, midx_v, mout)):
        for j in range(NCH):
            def fire(g, _):
                v = idx_s[pl.ds(j * CHUNK + g * 16, 16)]
                for i in range(16):
                    pltpu.async_copy(tab.at[v[i] >> 3],
                                     rows.at[g * 16 + i], sem)
                return _
            lax.fori_loop(0, CHUNK // 16, fire, 0)

            def drain(g, _):
                pltpu.make_async_copy(
                    tab.at[pl.ds(0, 16)],
                    rows.at[pl.ds(g * 16, 16)], sem).wait()
                return _
            lax.fori_loop(0, CHUNK // 16, drain, 0)

            def extract(g, _):
                v = idx_s[pl.ds(j * CHUNK + g * 16, 16)]
                for i in range(16):
                    k = g * 16 + i
                    sub = v[i] & (PACK - 1)
                    for c in range(D // 16):
                        ebuf[k, pl.ds(16 * c, 16)] = (
                            rows[k, sub, pl.ds(16 * c, 16)])
                return _
            lax.fori_loop(0, CHUNK // 16, extract, 0)
            pltpu.sync_copy(ebuf, out.at[pl.ds(base + j * CHUNK, CHUNK)])


def _sc_gather(utab, uidx, mtab, midx):
    mesh = plsc.VectorSubcoreMesh(core_axis_name="c", subcore_axis_name="s")
    fn = functools.partial(
        pl.kernel, mesh=mesh,
        compiler_params=pltpu.CompilerParams(use_tc_tiling_on_sc=True),
        out_type=(jax.ShapeDtypeStruct((B, D), jnp.float32),
                  jax.ShapeDtypeStruct((B, D), jnp.float32)),
        scratch_types=[
            pltpu.VMEM((BPW + 16,), jnp.int32),
            pltpu.VMEM((BPW + 16,), jnp.int32),
            pltpu.VMEM((CHUNK, PACK, D), jnp.float32),
            pltpu.VMEM((CHUNK, D), jnp.float32),
            pltpu.SemaphoreType.DMA,
        ],
    )(_sc_gather_body)
    return fn(utab, uidx, mtab, midx)


BLK = 1024


def _mlp_body(u_ref, m_ref, w1u_ref, w1m_ref, b1_ref,
              w2_ref, b2_ref, w3_ref, b3_ref, o_ref):
    h = jnp.dot(u_ref[...], w1u_ref[...], preferred_element_type=jnp.float32)
    h = h + jnp.dot(m_ref[...], w1m_ref[...],
                    preferred_element_type=jnp.float32)
    h = jnp.maximum(h + b1_ref[...], 0.0)
    h = jnp.dot(h, w2_ref[...], preferred_element_type=jnp.float32)
    h = jnp.maximum(h + b2_ref[...], 0.0)
    o_ref[...] = (jnp.dot(h, w3_ref[...], preferred_element_type=jnp.float32)
                  + b3_ref[...])


def _mlp(u_emb, m_emb, W1, b1, W2, b2, W3, b3):
    W1T = W1.T
    w1u = W1T[:D]
    w1m = W1T[D:]
    w2 = W2.T
    w3 = W3.T
    grid = (B // BLK,)
    return pl.pallas_call(
        _mlp_body,
        grid=grid,
        in_specs=[
            pl.BlockSpec((BLK, D), lambda i: (i, 0)),
            pl.BlockSpec((BLK, D), lambda i: (i, 0)),
            pl.BlockSpec((D, 128), lambda i: (0, 0)),
            pl.BlockSpec((D, 128), lambda i: (0, 0)),
            pl.BlockSpec((1, 128), lambda i: (0, 0)),
            pl.BlockSpec((128, D), lambda i: (0, 0)),
            pl.BlockSpec((1, D), lambda i: (0, 0)),
            pl.BlockSpec((D, 1), lambda i: (0, 0)),
            pl.BlockSpec((1, 1), lambda i: (0, 0)),
        ],
        out_specs=pl.BlockSpec((BLK, 1), lambda i: (i, 0)),
        out_shape=jax.ShapeDtypeStruct((B, 1), jnp.float32),
    )(u_emb, m_emb, w1u, w1m, b1.reshape(1, 128), w2, b2.reshape(1, D),
      w3, b3.reshape(1, 1))


def kernel(user, movie, user_table, movie_table, W1, b1, W2, b2, W3, b3):
    user = user.astype(jnp.int32)
    movie = movie.astype(jnp.int32)
    utab3 = user_table.reshape(user_table.shape[0] // PACK, PACK, D)
    mtab3 = movie_table.reshape(movie_table.shape[0] // PACK, PACK, D)
    u_emb, m_emb = _sc_gather(utab3, user, mtab3, movie)
    return _mlp(u_emb, m_emb, W1, b1, W2, b2, W3, b3)


# speedup vs baseline: 1.0340x; 1.0340x over previous
import functools

import jax
import jax.numpy as jnp
from jax import lax
from jax.experimental import pallas as pl
from jax.experimental.pallas import tpu as pltpu
from jax.experimental.pallas import tpu_sc as plsc

B = 16384
D = 64
PACK = 8
PW = PACK * D

_info = plsc.get_sparse_core_info()
NC, NS = _info.num_cores, _info.num_subcores
NW = NC * NS
BPW = B // NW
CHUNK = 64
NCH = BPW // CHUNK


def _sc_gather_body(utab, uidx, mtab, midx, uout, mout,
                    uidx_v, midx_v, rows, ebuf, sem):
    wid = lax.axis_index("s") * NC + lax.axis_index("c")
    base = wid * BPW
    pltpu.sync_copy(uidx.at[pl.ds(base, BPW)], uidx_v.at[pl.ds(0, BPW)])
    pltpu.sync_copy(midx.at[pl.ds(base, BPW)], midx_v.at[pl.ds(0, BPW)])
    def fire_u(v, i, k):
        pltpu.async_copy(utab.at[v[i] >> 3], rows.at[k], sem)

    def fire_m(v, i, k):
        t8 = pl.multiple_of((v[i] >> 3) * PACK, PACK)
        pltpu.async_copy(mtab.at[pl.ds(t8, PACK), :], rows.at[k], sem)

    for idx_s, out, fire_one in ((uidx_v, uout, fire_u),
                                 (midx_v, mout, fire_m)):
        for j in range(NCH):
            def fire(g, _):
                v = idx_s[pl.ds(j * CHUNK + g * 16, 16)]
                for i in range(16):
                    fire_one(v, i, g * 16 + i)
                return _
            lax.fori_loop(0, CHUNK // 16, fire, 0)

            def drain(g, _):
                pltpu.make_async_copy(
                    utab.at[pl.ds(0, 16)],
                    rows.at[pl.ds(g * 16, 16)], sem).wait()
                return _
            lax.fori_loop(0, CHUNK // 16, drain, 0)

            def extract(g, _):
                v = idx_s[pl.ds(j * CHUNK + g * 16, 16)]
                for i in range(16):
                    k = g * 16 + i
                    sub = v[i] & (PACK - 1)
                    for c in range(D // 16):
                        ebuf[k, pl.ds(16 * c, 16)] = (
                            rows[k, sub, pl.ds(16 * c, 16)])
                return _
            lax.fori_loop(0, CHUNK // 16, extract, 0)
            pltpu.sync_copy(ebuf, out.at[pl.ds(base + j * CHUNK, CHUNK)])


def _sc_gather(utab, uidx, mtab, midx):
    mesh = plsc.VectorSubcoreMesh(core_axis_name="c", subcore_axis_name="s")
    fn = functools.partial(
        pl.kernel, mesh=mesh,
        compiler_params=pltpu.CompilerParams(use_tc_tiling_on_sc=True),
        out_type=(jax.ShapeDtypeStruct((B, D), jnp.float32),
                  jax.ShapeDtypeStruct((B, D), jnp.float32)),
        scratch_types=[
            pltpu.VMEM((BPW + 16,), jnp.int32),
            pltpu.VMEM((BPW + 16,), jnp.int32),
            pltpu.VMEM((CHUNK, PACK, D), jnp.float32),
            pltpu.VMEM((CHUNK, D), jnp.float32),
            pltpu.SemaphoreType.DMA,
        ],
    )(_sc_gather_body)
    return fn(utab, uidx, mtab, midx)


BLK = 1024


def _mlp_body(u_ref, m_ref, w1u_ref, w1m_ref, b1_ref,
              w2_ref, b2_ref, w3_ref, b3_ref, o_ref):
    h = jnp.dot(u_ref[...], w1u_ref[...], preferred_element_type=jnp.float32)
    h = h + jnp.dot(m_ref[...], w1m_ref[...],
                    preferred_element_type=jnp.float32)
    h = jnp.maximum(h + b1_ref[...], 0.0)
    h = jnp.dot(h, w2_ref[...], preferred_element_type=jnp.float32)
    h = jnp.maximum(h + b2_ref[...], 0.0)
    o_ref[...] = (jnp.dot(h, w3_ref[...], preferred_element_type=jnp.float32)
                  + b3_ref[...])


def _mlp(u_emb, m_emb, W1, b1, W2, b2, W3, b3):
    W1T = W1.T
    w1u = W1T[:D]
    w1m = W1T[D:]
    w2 = W2.T
    w3 = W3.T
    grid = (B // BLK,)
    return pl.pallas_call(
        _mlp_body,
        grid=grid,
        in_specs=[
            pl.BlockSpec((BLK, D), lambda i: (i, 0)),
            pl.BlockSpec((BLK, D), lambda i: (i, 0)),
            pl.BlockSpec((D, 128), lambda i: (0, 0)),
            pl.BlockSpec((D, 128), lambda i: (0, 0)),
            pl.BlockSpec((1, 128), lambda i: (0, 0)),
            pl.BlockSpec((128, D), lambda i: (0, 0)),
            pl.BlockSpec((1, D), lambda i: (0, 0)),
            pl.BlockSpec((D, 1), lambda i: (0, 0)),
            pl.BlockSpec((1, 1), lambda i: (0, 0)),
        ],
        out_specs=pl.BlockSpec((BLK, 1), lambda i: (i, 0)),
        out_shape=jax.ShapeDtypeStruct((B, 1), jnp.float32),
    )(u_emb, m_emb, w1u, w1m, b1.reshape(1, 128), w2, b2.reshape(1, D),
      w3, b3.reshape(1, 1))


def kernel(user, movie, user_table, movie_table, W1, b1, W2, b2, W3, b3):
    user = user.astype(jnp.int32)
    movie = movie.astype(jnp.int32)
    utab3 = user_table.reshape(user_table.shape[0] // PACK, PACK, D)
    u_emb, m_emb = _sc_gather(utab3, user, movie_table, movie)
    return _mlp(u_emb, m_emb, W1, b1, W2, b2, W3, b3)


# speedup vs baseline: 1.0466x; 1.0122x over previous
import functools

import jax
import jax.numpy as jnp
from jax import lax
from jax.experimental import pallas as pl
from jax.experimental.pallas import tpu as pltpu
from jax.experimental.pallas import tpu_sc as plsc

B = 16384
D = 64
PACK = 8
PW = PACK * D

_info = plsc.get_sparse_core_info()
NC, NS = _info.num_cores, _info.num_subcores
NW = NC * NS
BPW = B // NW
CHUNK = 64
NCH = BPW // CHUNK


def _sc_gather_body(utab, uidx, mtab, midx, uout, mout,
                    uidx_v, midx_v, rows, ebuf, sem):
    wid = lax.axis_index("s") * NC + lax.axis_index("c")
    base = wid * BPW
    pltpu.sync_copy(uidx.at[pl.ds(base, BPW)], uidx_v.at[pl.ds(0, BPW)])
    pltpu.sync_copy(midx.at[pl.ds(base, BPW)], midx_v.at[pl.ds(0, BPW)])
    def fire_u(v, i, k):
        pltpu.async_copy(utab.at[v[i] >> 3], rows.at[k], sem)

    def fire_m(v, i, k):
        t8 = pl.multiple_of((v[i] >> 3) * PACK, PACK)
        pltpu.async_copy(mtab.at[pl.ds(t8, PACK), :], rows.at[k], sem)

    for idx_s, out, fire_one in ((uidx_v, uout, fire_u),
                                 (midx_v, mout, fire_m)):
        for j in range(NCH):
            def fire(g, _):
                v = idx_s[pl.ds(j * CHUNK + g * 16, 16)]
                for i in range(16):
                    fire_one(v, i, g * 16 + i)
                return _
            lax.fori_loop(0, CHUNK // 16, fire, 0)

            def drain(g, _):
                pltpu.make_async_copy(
                    utab.at[pl.ds(0, 16)],
                    rows.at[pl.ds(g * 16, 16)], sem).wait()
                return _
            lax.fori_loop(0, CHUNK // 16, drain, 0)

            def extract(g, _):
                v = idx_s[pl.ds(j * CHUNK + g * 16, 16)]
                for i in range(16):
                    k = g * 16 + i
                    sub = v[i] & (PACK - 1)
                    for c in range(D // 16):
                        ebuf[k, pl.ds(16 * c, 16)] = (
                            rows[k, sub, pl.ds(16 * c, 16)])
                return _
            lax.fori_loop(0, CHUNK // 16, extract, 0)
            pltpu.sync_copy(ebuf, out.at[pl.ds(base + j * CHUNK, CHUNK)])


def _sc_gather(utab, uidx, mtab, midx):
    mesh = plsc.VectorSubcoreMesh(core_axis_name="c", subcore_axis_name="s")
    fn = functools.partial(
        pl.kernel, mesh=mesh,
        compiler_params=pltpu.CompilerParams(use_tc_tiling_on_sc=True),
        out_type=(jax.ShapeDtypeStruct((B, D), jnp.float32),
                  jax.ShapeDtypeStruct((B, D), jnp.float32)),
        scratch_types=[
            pltpu.VMEM((BPW + 16,), jnp.int32),
            pltpu.VMEM((BPW + 16,), jnp.int32),
            pltpu.VMEM((CHUNK, PACK, D), jnp.float32),
            pltpu.VMEM((CHUNK, D), jnp.float32),
            pltpu.SemaphoreType.DMA,
        ],
    )(_sc_gather_body)
    return fn(utab, uidx, mtab, midx)


BLK = 2048


def _mlp_body(u_ref, m_ref, w1u_ref, w1m_ref, b1_ref,
              w2_ref, b2_ref, w3_ref, b3_ref, o_ref):
    h = jnp.dot(u_ref[...], w1u_ref[...], preferred_element_type=jnp.float32)
    h = h + jnp.dot(m_ref[...], w1m_ref[...],
                    preferred_element_type=jnp.float32)
    h = jnp.maximum(h + b1_ref[...], 0.0)
    h = jnp.dot(h, w2_ref[...], preferred_element_type=jnp.float32)
    h = jnp.maximum(h + b2_ref[...], 0.0)
    o_ref[...] = (jnp.dot(h, w3_ref[...], preferred_element_type=jnp.float32)
                  + b3_ref[...])


def _mlp(u_emb, m_emb, W1, b1, W2, b2, W3, b3):
    W1T = W1.T
    w1u = W1T[:D]
    w1m = W1T[D:]
    w2 = W2.T
    w3 = W3.T
    grid = (B // BLK,)
    return pl.pallas_call(
        _mlp_body,
        grid=grid,
        in_specs=[
            pl.BlockSpec((BLK, D), lambda i: (i, 0)),
            pl.BlockSpec((BLK, D), lambda i: (i, 0)),
            pl.BlockSpec((D, 128), lambda i: (0, 0)),
            pl.BlockSpec((D, 128), lambda i: (0, 0)),
            pl.BlockSpec((1, 128), lambda i: (0, 0)),
            pl.BlockSpec((128, D), lambda i: (0, 0)),
            pl.BlockSpec((1, D), lambda i: (0, 0)),
            pl.BlockSpec((D, 1), lambda i: (0, 0)),
            pl.BlockSpec((1, 1), lambda i: (0, 0)),
        ],
        out_specs=pl.BlockSpec((BLK, 1), lambda i: (i, 0)),
        out_shape=jax.ShapeDtypeStruct((B, 1), jnp.float32),
    )(u_emb, m_emb, w1u, w1m, b1.reshape(1, 128), w2, b2.reshape(1, D),
      w3, b3.reshape(1, 1))


def kernel(user, movie, user_table, movie_table, W1, b1, W2, b2, W3, b3):
    user = user.astype(jnp.int32)
    movie = movie.astype(jnp.int32)
    utab3 = user_table.reshape(user_table.shape[0] // PACK, PACK, D)
    u_emb, m_emb = _sc_gather(utab3, user, movie_table, movie)
    return _mlp(u_emb, m_emb, W1, b1, W2, b2, W3, b3)
